# baseline (speedup 1.0000x reference)
import jax
import jax.numpy as jnp
from jax.experimental import pallas as pl
from jax.experimental.pallas import tpu as pltpu

_NUM_EMBEDDINGS = 1024
_EMBEDDING_DIM = 64
_BETA = 0.25
_BLK = 4096
_UNROLL = 8
_N_TOKENS = 16 * 32 * 32


def _vq_block_kernel(x_ref, emb_ref, out_ref, idx_ref, loss_ref,
                     nemb_ref, e_n_ref, embq_ref, acc_ref):
    @pl.when(pl.program_id(0) == 0)
    def _init():
        emb = emb_ref[...]
        nemb = emb * (-2.0)
        nemb_ref[...] = nemb
        e_n_ref[...] = jnp.sum(nemb * nemb, axis=0, keepdims=True) * 0.25
        embq_ref[...] = emb.astype(jnp.bfloat16)
        acc_ref[...] = jnp.zeros_like(acc_ref)

    x = x_ref[...]
    nemb = nemb_ref[...]
    embq = embq_ref[...]
    e_n = e_n_ref[...]
    sub = _BLK // _UNROLL
    idx_parts = []
    diff_parts = []
    for u in range(_UNROLL):
        xs = x[u * sub:(u + 1) * sub, :]
        sim2 = jnp.dot(xs, nemb, preferred_element_type=jnp.float32)
        d1 = jnp.sum(xs * xs, axis=1, keepdims=True)
        dist = d1 + (e_n + sim2)
        idx = jnp.argmin(dist, axis=1).astype(jnp.int32)
        onehot = (jax.lax.broadcasted_iota(jnp.int32, (sub, _NUM_EMBEDDINGS), 1)
                  == idx[:, None]).astype(jnp.bfloat16)
        q = jax.lax.dot_general(onehot, embq,
                                dimension_numbers=(((1,), (1,)), ((), ())),
                                preferred_element_type=jnp.float32)
        idx_parts.append(idx[:, None])
        diff_parts.append(q - xs)
    diff = jnp.concatenate(diff_parts, axis=0)
    out_ref[...] = x + diff
    idx_ref[...] = jnp.concatenate(idx_parts, axis=0)
    acc_ref[...] += jnp.sum(diff * diff, axis=0, keepdims=True)

    @pl.when(pl.program_id(0) == (_N_TOKENS // _BLK) - 1)
    def _fin():
        scale = (1.0 + _BETA) / (_N_TOKENS * _EMBEDDING_DIM)
        loss_ref[...] = jnp.sum(acc_ref[...], axis=1, keepdims=True) * scale


def kernel(x, embeddings):
    input_shape = x.shape
    xf = x.reshape(-1, _EMBEDDING_DIM)
    n = xf.shape[0]
    grid = (n // _BLK,)
    out, idx, loss = pl.pallas_call(
        _vq_block_kernel,
        grid=grid,
        in_specs=[
            pl.BlockSpec((_BLK, _EMBEDDING_DIM), lambda i: (i, 0)),
            pl.BlockSpec((_EMBEDDING_DIM, _NUM_EMBEDDINGS), lambda i: (0, 0)),
        ],
        out_specs=[
            pl.BlockSpec((_BLK, _EMBEDDING_DIM), lambda i: (i, 0)),
            pl.BlockSpec((_BLK, 1), lambda i: (i, 0)),
            pl.BlockSpec((1, 1), lambda i: (0, 0)),
        ],
        out_shape=[
            jax.ShapeDtypeStruct((n, _EMBEDDING_DIM), jnp.float32),
            jax.ShapeDtypeStruct((n, 1), jnp.int32),
            jax.ShapeDtypeStruct((1, 1), jnp.float32),
        ],
        scratch_shapes=[
            pltpu.VMEM((_EMBEDDING_DIM, _NUM_EMBEDDINGS), jnp.float32),
            pltpu.VMEM((1, _NUM_EMBEDDINGS), jnp.float32),
            pltpu.VMEM((_EMBEDDING_DIM, _NUM_EMBEDDINGS), jnp.bfloat16),
            pltpu.VMEM((1, _EMBEDDING_DIM), jnp.float32),
        ],
    )(xf, embeddings)
    quantized_ste = out.reshape(input_shape)
    encoding_indices = idx.reshape(-1, 4 * 4)
    return quantized_ste, encoding_indices, loss.reshape(())


# speedup vs baseline: 1.0238x; 1.0238x over previous
import jax
import jax.numpy as jnp
from jax.experimental import pallas as pl
from jax.experimental.pallas import tpu as pltpu

_NUM_EMBEDDINGS = 1024
_EMBEDDING_DIM = 64
_BETA = 0.25
_BLK = 8192
_UNROLL = 16
_N_TOKENS = 16 * 32 * 32


def _vq_block_kernel(x_ref, emb_ref, out_ref, idx_ref, loss_ref,
                     nemb_ref, e_n_ref, embq_ref, acc_ref):
    @pl.when(pl.program_id(0) == 0)
    def _init():
        emb = emb_ref[...]
        nemb = emb * (-2.0)
        nemb_ref[...] = nemb
        e_n_ref[...] = jnp.sum(nemb * nemb, axis=0, keepdims=True) * 0.25
        embq_ref[...] = emb.astype(jnp.bfloat16)
        acc_ref[...] = jnp.zeros_like(acc_ref)

    x = x_ref[...]
    nemb = nemb_ref[...]
    embq = embq_ref[...]
    e_n = e_n_ref[...]
    sub = _BLK // _UNROLL
    idx_parts = []
    diff_parts = []
    for u in range(_UNROLL):
        xs = x[u * sub:(u + 1) * sub, :]
        sim2 = jnp.dot(xs, nemb, preferred_element_type=jnp.float32)
        d1 = jnp.sum(xs * xs, axis=1, keepdims=True)
        dist = d1 + (e_n + sim2)
        idx = jnp.argmin(dist, axis=1).astype(jnp.int32)
        onehot = (jax.lax.broadcasted_iota(jnp.int32, (sub, _NUM_EMBEDDINGS), 1)
                  == idx[:, None]).astype(jnp.bfloat16)
        q = jax.lax.dot_general(onehot, embq,
                                dimension_numbers=(((1,), (1,)), ((), ())),
                                preferred_element_type=jnp.float32)
        idx_parts.append(idx[:, None])
        diff_parts.append(q - xs)
    diff = jnp.concatenate(diff_parts, axis=0)
    out_ref[...] = (x + diff).reshape(out_ref.shape)
    idx_ref[...] = jnp.concatenate(idx_parts, axis=0)
    acc_ref[...] += jnp.sum(diff * diff, axis=0, keepdims=True)

    @pl.when(pl.program_id(0) == (_N_TOKENS // _BLK) - 1)
    def _fin():
        scale = (1.0 + _BETA) / (_N_TOKENS * _EMBEDDING_DIM)
        loss_ref[...] = jnp.sum(acc_ref[...], axis=1, keepdims=True) * scale


def kernel(x, embeddings):
    input_shape = x.shape
    xf = x.reshape(-1, _EMBEDDING_DIM)
    n = xf.shape[0]
    grid = (n // _BLK,)
    out, idx, loss = pl.pallas_call(
        _vq_block_kernel,
        grid=grid,
        in_specs=[
            pl.BlockSpec((_BLK, _EMBEDDING_DIM), lambda i: (i, 0)),
            pl.BlockSpec((_EMBEDDING_DIM, _NUM_EMBEDDINGS), lambda i: (0, 0)),
        ],
        out_specs=[
            pl.BlockSpec((_BLK // 1024, 32, 32, _EMBEDDING_DIM),
                         lambda i: (i, 0, 0, 0)),
            pl.BlockSpec((_BLK, 1), lambda i: (i, 0)),
            pl.BlockSpec((1, 1), lambda i: (0, 0)),
        ],
        out_shape=[
            jax.ShapeDtypeStruct(input_shape, jnp.float32),
            jax.ShapeDtypeStruct((n, 1), jnp.int32),
            jax.ShapeDtypeStruct((1, 1), jnp.float32),
        ],
        scratch_shapes=[
            pltpu.VMEM((_EMBEDDING_DIM, _NUM_EMBEDDINGS), jnp.float32),
            pltpu.VMEM((1, _NUM_EMBEDDINGS), jnp.float32),
            pltpu.VMEM((_EMBEDDING_DIM, _NUM_EMBEDDINGS), jnp.bfloat16),
            pltpu.VMEM((1, _EMBEDDING_DIM), jnp.float32),
        ],
    )(xf, embeddings)
    quantized_ste = out
    encoding_indices = idx.reshape(-1, 4 * 4)
    return quantized_ste, encoding_indices, loss.reshape(())
